# baseline (device time: 28060 ns/iter reference)
import jax
import jax.numpy as jnp
from jax import lax
from jax.experimental import pallas as pl
from jax.experimental.pallas import tpu as pltpu


def kernel(Q, K, V):
    b, sq, h, d = Q.shape
    skv = K.shape[1]
    hd = h * d
    bh = b // 2
    scale = d ** -0.5

    my_x = lax.axis_index("x")
    Qh = lax.dynamic_slice_in_dim(Q, my_x * bh, bh, axis=0)
    Kh = lax.dynamic_slice_in_dim(K, my_x * bh, bh, axis=0).astype(jnp.bfloat16)
    Vh = lax.dynamic_slice_in_dim(V, my_x * bh, bh, axis=0).astype(jnp.bfloat16)

    def body(q_ref, k_ref, v_ref, out_ref, o_buf, ml_buf,
             kbuf, vbuf, copy_sems, send_sems, recv_sems):
        my_x = lax.axis_index("x")
        my_y = lax.axis_index("y")
        y_peer = (my_x, 1 - my_y)
        x_peer = (1 - my_x, my_y)
        dg_peer = (1 - my_x, 1 - my_y)
        peers = (y_peer, x_peer, dg_peer)

        barrier = pltpu.get_barrier_semaphore()
        for peer in peers:
            pl.semaphore_signal(
                barrier, inc=1, device_id=peer,
                device_id_type=pl.DeviceIdType.MESH,
            )
        pl.semaphore_wait(barrier, 3)

        iota = lax.broadcasted_iota
        ETb = (iota(jnp.int32, (h, hd), 1) // d
               == iota(jnp.int32, (h, hd), 0))
        ETm = ETb.astype(jnp.float32)
        ETb = ETb.astype(jnp.bfloat16)

        def start_chunk(bi):
            slot = bi % 2
            ck = pltpu.make_async_copy(
                k_ref.at[bi], kbuf.at[slot], copy_sems.at[slot]
            )
            cv = pltpu.make_async_copy(
                v_ref.at[bi], vbuf.at[slot], copy_sems.at[2 + slot]
            )
            ck.start()
            cv.start()
            return ck, cv

        pending = {0: start_chunk(0)}
        ms, ls, os_ = [], [], []
        for bi in range(bh):
            if bi + 1 < bh:
                pending[bi + 1] = start_chunk(bi + 1)
            ck, cv = pending.pop(bi)
            ck.wait()
            cv.wait()
            slot = bi % 2
            kb2 = kbuf[slot].reshape(skv, hd)
            vb2 = vbuf[slot].reshape(skv, hd)
            qb = q_ref[bi, 0].astype(jnp.bfloat16)
            qrow = jnp.tile(qb, (1, h)) * ETb
            qblk = qrow.T
            s_kh = lax.dot_general(
                kb2, qblk, (((1,), (0,)), ((), ())),
                preferred_element_type=jnp.float32,
            ) * scale
            m = jnp.max(s_kh, axis=0, keepdims=True)
            p = jnp.exp(s_kh - m)
            l = jnp.sum(p, axis=0, keepdims=True)
            o_t = lax.dot_general(
                p.astype(jnp.bfloat16), vb2, (((0,), (0,)), ((), ())),
                preferred_element_type=jnp.float32,
            )
            o = jnp.sum((o_t * ETm).reshape(h, h, d), axis=1)
            ms.append(m)
            ls.append(l)
            os_.append(o[None])
        m_loc = jnp.concatenate(ms, axis=0)
        l_loc = jnp.concatenate(ls, axis=0)
        o_loc = jnp.concatenate(os_, axis=0)

        o_buf[0] = o_loc
        ml_buf[0, 0] = m_loc
        ml_buf[0, 1] = l_loc

        rdmas = []
        for idx, peer in enumerate(peers):
            slot = idx + 1
            rdmas.append(pltpu.make_async_remote_copy(
                src_ref=o_buf.at[0], dst_ref=o_buf.at[slot],
                send_sem=send_sems.at[idx], recv_sem=recv_sems.at[idx],
                device_id=peer, device_id_type=pl.DeviceIdType.MESH,
            ))
            rdmas.append(pltpu.make_async_remote_copy(
                src_ref=ml_buf.at[0], dst_ref=ml_buf.at[slot],
                send_sem=send_sems.at[3 + idx], recv_sem=recv_sems.at[3 + idx],
                device_id=peer, device_id_type=pl.DeviceIdType.MESH,
            ))
        for r in rdmas:
            r.start()
        for r in rdmas:
            r.wait()

        def merge(sa, sb):
            m_a, l_a, o_a = ml_buf[sa, 0], ml_buf[sa, 1], o_buf[sa]
            m_b, l_b, o_b = ml_buf[sb, 0], ml_buf[sb, 1], o_buf[sb]
            m_t = jnp.maximum(m_a, m_b)
            a_a = jnp.exp(m_a - m_t)
            a_b = jnp.exp(m_b - m_t)
            l_t = l_a * a_a + l_b * a_b
            return (o_a * a_a[:, :, None] + o_b * a_b[:, :, None]) \
                / l_t[:, :, None]

        out_ref[pl.ds(my_x * bh, bh), 0] = merge(0, 1)
        out_ref[pl.ds((1 - my_x) * bh, bh), 0] = merge(2, 3)

    return pl.pallas_call(
        body,
        out_shape=jax.ShapeDtypeStruct((b, sq, h, d), jnp.float32),
        in_specs=[
            pl.BlockSpec(memory_space=pltpu.VMEM),
            pl.BlockSpec(memory_space=pl.ANY),
            pl.BlockSpec(memory_space=pl.ANY),
        ],
        out_specs=pl.BlockSpec(memory_space=pltpu.VMEM),
        scratch_shapes=[
            pltpu.VMEM((4, bh, h, d), jnp.float32),
            pltpu.VMEM((4, 2, bh, h), jnp.float32),
            pltpu.VMEM((2, skv, h, d), jnp.bfloat16),
            pltpu.VMEM((2, skv, h, d), jnp.bfloat16),
            pltpu.SemaphoreType.DMA((4,)),
            pltpu.SemaphoreType.DMA((6,)),
            pltpu.SemaphoreType.DMA((6,)),
        ],
        compiler_params=pltpu.CompilerParams(
            collective_id=0,
            vmem_limit_bytes=100 * 1024 * 1024,
        ),
    )(Qh, Kh, Vh)


# device time: 27570 ns/iter; 1.0178x vs baseline; 1.0178x over previous
import jax
import jax.numpy as jnp
from jax import lax
from jax.experimental import pallas as pl
from jax.experimental.pallas import tpu as pltpu


def kernel(Q, K, V):
    b, sq, h, d = Q.shape
    skv = K.shape[1]
    hd = h * d
    bh = b // 2
    scale = d ** -0.5

    my_x = lax.axis_index("x")
    Qh = lax.dynamic_slice_in_dim(Q, my_x * bh, bh, axis=0)
    Kh = lax.dynamic_slice_in_dim(K, my_x * bh, bh, axis=0).astype(jnp.bfloat16)
    Vh = lax.dynamic_slice_in_dim(V, my_x * bh, bh, axis=0).astype(jnp.bfloat16)

    def body(q_ref, k_ref, v_ref, out_ref, o_buf, ml_buf,
             send_sems, recv_sems):
        my_x = lax.axis_index("x")
        my_y = lax.axis_index("y")
        y_peer = (my_x, 1 - my_y)
        x_peer = (1 - my_x, my_y)
        dg_peer = (1 - my_x, 1 - my_y)
        peers = (y_peer, x_peer, dg_peer)

        barrier = pltpu.get_barrier_semaphore()
        for peer in peers:
            pl.semaphore_signal(
                barrier, inc=1, device_id=peer,
                device_id_type=pl.DeviceIdType.MESH,
            )
        pl.semaphore_wait(barrier, 3)

        iota = lax.broadcasted_iota
        ETb = (iota(jnp.int32, (h, hd), 1) // d
               == iota(jnp.int32, (h, hd), 0))
        ETm = ETb.astype(jnp.float32)
        ETb = ETb.astype(jnp.bfloat16)

        k2 = k_ref[...].reshape(bh * skv, hd)
        v2 = v_ref[...].reshape(bh * skv, hd)

        ms, ls, os_ = [], [], []
        for bi in range(bh):
            kb2 = k2[bi * skv:(bi + 1) * skv]
            vb2 = v2[bi * skv:(bi + 1) * skv]
            qb = q_ref[bi, 0].astype(jnp.bfloat16)
            qrow = jnp.tile(qb, (1, h)) * ETb
            qblk = qrow.T
            s_kh = lax.dot_general(
                kb2, qblk, (((1,), (0,)), ((), ())),
                preferred_element_type=jnp.float32,
            ) * scale
            m = jnp.max(s_kh, axis=0, keepdims=True)
            p = jnp.exp(s_kh - m)
            l = jnp.sum(p, axis=0, keepdims=True)
            o_t = lax.dot_general(
                p.astype(jnp.bfloat16), vb2, (((0,), (0,)), ((), ())),
                preferred_element_type=jnp.float32,
            )
            o = jnp.sum((o_t * ETm).reshape(h, h, d), axis=1)
            ms.append(m)
            ls.append(l)
            os_.append(o[None])
        m_loc = jnp.concatenate(ms, axis=0)
        l_loc = jnp.concatenate(ls, axis=0)
        o_loc = jnp.concatenate(os_, axis=0)

        o_buf[0] = o_loc
        ml_buf[0, 0] = m_loc
        ml_buf[0, 1] = l_loc

        rdmas = []
        for idx, peer in enumerate(peers):
            slot = idx + 1
            rdmas.append(pltpu.make_async_remote_copy(
                src_ref=o_buf.at[0], dst_ref=o_buf.at[slot],
                send_sem=send_sems.at[idx], recv_sem=recv_sems.at[idx],
                device_id=peer, device_id_type=pl.DeviceIdType.MESH,
            ))
            rdmas.append(pltpu.make_async_remote_copy(
                src_ref=ml_buf.at[0], dst_ref=ml_buf.at[slot],
                send_sem=send_sems.at[3 + idx], recv_sem=recv_sems.at[3 + idx],
                device_id=peer, device_id_type=pl.DeviceIdType.MESH,
            ))
        for r in rdmas:
            r.start()
        for r in rdmas:
            r.wait()

        def merge(sa, sb):
            m_a, l_a, o_a = ml_buf[sa, 0], ml_buf[sa, 1], o_buf[sa]
            m_b, l_b, o_b = ml_buf[sb, 0], ml_buf[sb, 1], o_buf[sb]
            m_t = jnp.maximum(m_a, m_b)
            a_a = jnp.exp(m_a - m_t)
            a_b = jnp.exp(m_b - m_t)
            l_t = l_a * a_a + l_b * a_b
            return (o_a * a_a[:, :, None] + o_b * a_b[:, :, None]) \
                / l_t[:, :, None]

        out_ref[pl.ds(my_x * bh, bh), 0] = merge(0, 1)
        out_ref[pl.ds((1 - my_x) * bh, bh), 0] = merge(2, 3)

    return pl.pallas_call(
        body,
        out_shape=jax.ShapeDtypeStruct((b, sq, h, d), jnp.float32),
        in_specs=[pl.BlockSpec(memory_space=pltpu.VMEM)] * 3,
        out_specs=pl.BlockSpec(memory_space=pltpu.VMEM),
        scratch_shapes=[
            pltpu.VMEM((4, bh, h, d), jnp.float32),
            pltpu.VMEM((4, 2, bh, h), jnp.float32),
            pltpu.SemaphoreType.DMA((6,)),
            pltpu.SemaphoreType.DMA((6,)),
        ],
        compiler_params=pltpu.CompilerParams(
            collective_id=0,
            vmem_limit_bytes=100 * 1024 * 1024,
            allow_input_fusion=[False, True, True],
        ),
    )(Qh, Kh, Vh)
